# Initial kernel scaffold; baseline (speedup 1.0000x reference)
import jax
import jax.numpy as jnp
from jax.experimental import pallas as pl


def kernel(x, edge_index, first_edge, candidate_edges, t, gin_W1, gin_b1, gin_W2, gin_b2, gin_eps, bn_gamma, bn_beta, t_W, t_P, t_aff_w, t_aff_b, ep_W1, ep_b1, ep_W2, ep_b2, oh_W1, oh_b1, oh_W2, oh_b2):
    raise NotImplementedError("write your pallas kernel here")


# speedup vs baseline: 6.9632x; 6.9632x over previous
import functools

import jax
import jax.numpy as jnp
from jax import lax
from jax.experimental import pallas as pl
from jax.experimental.pallas import tpu as pltpu
from jax.experimental.pallas import tpu_sc as plsc

N = 10000
E = 320000
D = 128
L = 3
C = 512

NC = 2
NS = 16
NW = NC * NS
EPW = E // NW
CH = 80
NCHUNK = EPW // CH
ROWS_PER_TILE = 632
NPAD = NS * ROWS_PER_TILE

G = 1280
GPW = G // NW

RB = 400
NB = N // RB

_HI = lax.Precision.HIGHEST


def _build_seg_sum():
    mesh = plsc.VectorSubcoreMesh(core_axis_name="c", subcore_axis_name="s", num_cores=NC, num_subcores=NS)

    @functools.partial(
        pl.kernel,
        out_type=jax.ShapeDtypeStruct((NC, NPAD, D), jnp.float32),
        mesh=mesh,
        scratch_types=[
            pltpu.VMEM_SHARED((NPAD, D), jnp.float32),
            pltpu.VMEM((2, CH), jnp.int32),
            pltpu.VMEM((2, CH), jnp.int32),
            pltpu.VMEM((CH, D), jnp.float32),
            pltpu.VMEM((CH, D), jnp.float32),
            pltpu.VMEM((32, D), jnp.float32),
            pltpu.SemaphoreType.DMA,
            pltpu.SemaphoreType.DMA,
            pltpu.SemaphoreType.DMA,
            pltpu.SemaphoreType.DMA,
        ],
    )
    def seg_sum(h_hbm, eidx_hbm, out_hbm,
                acc, ib0, ib1, rows0, rows1, zbuf, is0, is1, gs0, gs1):
        c = lax.axis_index("c")
        s = lax.axis_index("s")
        w = s * NC + c

        zero16 = jnp.zeros((16,), jnp.float32)

        def zb_body(k, carry):
            zbuf[k // 8, pl.ds((k % 8) * 16, 16)] = zero16
            return carry

        lax.fori_loop(0, 32 * 8, zb_body, 0)
        r0 = s * ROWS_PER_TILE
        for tblk in range(ROWS_PER_TILE // 32 - 1):
            pltpu.sync_copy(zbuf, acc.at[pl.ds(r0 + tblk * 32, 32), :])
        pltpu.sync_copy(zbuf.at[pl.ds(0, ROWS_PER_TILE % 32 or 32), :],
                        acc.at[pl.ds(r0 + (ROWS_PER_TILE // 32 - 1) * 32,
                                     ROWS_PER_TILE % 32 or 32), :])
        plsc.subcore_barrier()

        pltpu.sync_copy(eidx_hbm.at[w, 0], ib0)
        pltpu.async_copy(eidx_hbm.at[w, 1], ib1, is1)
        pltpu.async_copy(h_hbm.at[ib0.at[0]], rows0, gs0)

        def body(k, carry):
            j0 = 2 * k
            pltpu.make_async_copy(h_hbm.at[ib0.at[0]], rows0, gs0).wait()
            pltpu.make_async_copy(eidx_hbm.at[w, j0 + 1], ib1, is1).wait()
            pltpu.async_copy(h_hbm.at[ib1.at[0]], rows1, gs1)
            pltpu.sync_copy(rows0, acc.at[ib0.at[1]], add=True)
            pltpu.async_copy(eidx_hbm.at[w, j0 + 2], ib0, is0)
            pltpu.make_async_copy(h_hbm.at[ib1.at[0]], rows1, gs1).wait()
            pltpu.make_async_copy(eidx_hbm.at[w, j0 + 2], ib0, is0).wait()
            pltpu.async_copy(h_hbm.at[ib0.at[0]], rows0, gs0)
            pltpu.sync_copy(rows1, acc.at[ib1.at[1]], add=True)
            pltpu.async_copy(eidx_hbm.at[w, j0 + 3], ib1, is1)
            return carry

        lax.fori_loop(0, (NCHUNK - 1) // 2, body, 0)
        pltpu.make_async_copy(h_hbm.at[ib0.at[0]], rows0, gs0).wait()
        pltpu.make_async_copy(eidx_hbm.at[w, NCHUNK], ib1, is1).wait()
        pltpu.sync_copy(rows0, acc.at[ib0.at[1]], add=True)

        plsc.subcore_barrier()

        pltpu.sync_copy(acc.at[pl.ds(r0, ROWS_PER_TILE), :],
                        out_hbm.at[c, pl.ds(r0, ROWS_PER_TILE), :])

    return seg_sum


_seg_sum = _build_seg_sum()


def _build_gather():
    mesh = plsc.VectorSubcoreMesh(core_axis_name="c", subcore_axis_name="s", num_cores=NC, num_subcores=NS)

    @functools.partial(
        pl.kernel,
        out_type=jax.ShapeDtypeStruct((G, D), jnp.float32),
        mesh=mesh,
        scratch_types=[
            pltpu.VMEM((GPW,), jnp.int32),
            pltpu.VMEM((GPW, D), jnp.float32),
            pltpu.SemaphoreType.DMA,
        ],
    )
    def gather_rows(h_hbm, idx_hbm, out_hbm, idxv, rowsv, sem):
        c = lax.axis_index("c")
        s = lax.axis_index("s")
        w = s * NC + c
        base = w * GPW
        pltpu.sync_copy(idx_hbm.at[pl.ds(base, GPW)], idxv)
        pltpu.async_copy(h_hbm.at[idxv], rowsv, sem).wait()
        pltpu.sync_copy(rowsv, out_hbm.at[pl.ds(base, GPW), :])

    return gather_rows


_gather_rows = _build_gather()


def _mlp_body(eps_ref, h_ref, agg_ref, w1_ref, b1_ref, w2_ref, b2_ref,
              z_ref, st_ref, acc_ref):
    i = pl.program_id(0)
    u = eps_ref[0] * h_ref[...] + agg_ref[0] + agg_ref[1]
    a1 = jnp.maximum(
        jnp.dot(u, w1_ref[...], preferred_element_type=jnp.float32,
                precision=_HI) + b1_ref[...], 0.0)
    z = jnp.dot(a1, w2_ref[...], preferred_element_type=jnp.float32,
                precision=_HI) + b2_ref[...]
    z_ref[...] = z
    ps = jnp.sum(z, axis=0, keepdims=True)
    pq = jnp.sum(z * z, axis=0, keepdims=True)
    stk = jnp.concatenate([ps, pq], axis=0)

    @pl.when(i == 0)
    def _():
        acc_ref[...] = stk

    @pl.when(i > 0)
    def _():
        acc_ref[...] = acc_ref[...] + stk

    @pl.when(i == NB - 1)
    def _():
        st_ref[...] = acc_ref[...]


def _mlp(h, agg, w1, b1, w2, b2, eps1):
    return pl.pallas_call(
        _mlp_body,
        grid=(NB,),
        in_specs=[
            pl.BlockSpec(memory_space=pltpu.SMEM),
            pl.BlockSpec((RB, D), lambda i: (i, 0)),
            pl.BlockSpec((NC, RB, D), lambda i: (0, i, 0)),
            pl.BlockSpec((D, D), lambda i: (0, 0)),
            pl.BlockSpec((1, D), lambda i: (0, 0)),
            pl.BlockSpec((D, D), lambda i: (0, 0)),
            pl.BlockSpec((1, D), lambda i: (0, 0)),
        ],
        out_specs=[
            pl.BlockSpec((RB, D), lambda i: (i, 0)),
            pl.BlockSpec((2, D), lambda i: (0, 0)),
        ],
        out_shape=[
            jax.ShapeDtypeStruct((N, D), jnp.float32),
            jax.ShapeDtypeStruct((2, D), jnp.float32),
        ],
        scratch_shapes=[pltpu.VMEM((2, D), jnp.float32)],
    )(eps1, h, agg, w1, b1, w2, b2)


def _bn_body(z_ref, h_ref, st_ref, g_ref, b_ref, o_ref):
    mean = st_ref[0:1, :] * (1.0 / N)
    ex2 = st_ref[1:2, :] * (1.0 / N)
    var = ex2 - mean * mean
    inv = lax.rsqrt(var + 1e-5)
    o_ref[...] = (z_ref[...] - mean) * (inv * g_ref[...]) + b_ref[...] + h_ref[...]


def _bnres(z, h, st, gamma, beta):
    return pl.pallas_call(
        _bn_body,
        grid=(NB,),
        in_specs=[
            pl.BlockSpec((RB, D), lambda i: (i, 0)),
            pl.BlockSpec((RB, D), lambda i: (i, 0)),
            pl.BlockSpec((2, D), lambda i: (0, 0)),
            pl.BlockSpec((1, D), lambda i: (0, 0)),
            pl.BlockSpec((1, D), lambda i: (0, 0)),
        ],
        out_specs=pl.BlockSpec((RB, D), lambda i: (i, 0)),
        out_shape=jax.ShapeDtypeStruct((N, D), jnp.float32),
    )(z, h, st, gamma, beta)


def _score_body(t_ref, rows_ref, tw_ref, tp_ref, taw_ref, tab_ref,
                epw1_ref, epb1_ref, epw2_ref, epb2_ref,
                ohw1_ref, ohb1_ref, ohw2_ref, ohb2_ref,
                pl_ref, ol_ref):
    tf = t_ref[0]
    ang = tw_ref[...] * tf + tp_ref[...]
    aff = taw_ref[...] * tf + tab_ref[...]
    temb = jnp.concatenate([jnp.sin(ang), jnp.cos(ang), aff], axis=1)
    ha = rows_ref[0:1, :]
    hb = rows_ref[1:2, :]
    uv = jnp.concatenate([ha + hb, jnp.abs(ha - hb)], axis=1)
    a_rows = rows_ref[8:8 + C, :]
    b_rows = rows_ref[520:520 + C, :]
    s_xy = a_rows + b_rows
    d_xy = jnp.abs(a_rows - b_rows)

    def head(w1_ref, b1_ref):
        const = (jnp.dot(uv, w1_ref[0:256, :], preferred_element_type=jnp.float32,
                         precision=_HI)
                 + jnp.dot(temb, w1_ref[512:704, :],
                           preferred_element_type=jnp.float32, precision=_HI)
                 + b1_ref[...])
        act = (jnp.dot(s_xy, w1_ref[256:384, :], preferred_element_type=jnp.float32,
                       precision=_HI)
               + jnp.dot(d_xy, w1_ref[384:512, :],
                         preferred_element_type=jnp.float32, precision=_HI)
               + const)
        return jnp.maximum(act, 0.0)

    hp = head(epw1_ref, epb1_ref)
    pl_ref[...] = jnp.dot(hp, epw2_ref[...], preferred_element_type=jnp.float32,
                          precision=_HI) + epb2_ref[...]
    ho = head(ohw1_ref, ohb1_ref)
    ol_ref[...] = jnp.dot(ho, ohw2_ref[...], preferred_element_type=jnp.float32,
                          precision=_HI) + ohb2_ref[...]


def _score(tf, rows, t_w, t_p, t_aw, t_ab, epw1, epb1, epw2, epb2,
           ohw1, ohb1, ohw2, ohb2):
    return pl.pallas_call(
        _score_body,
        in_specs=[pl.BlockSpec(memory_space=pltpu.SMEM)] + [pl.BlockSpec()] * 13,
        out_specs=[pl.BlockSpec(), pl.BlockSpec()],
        out_shape=[
            jax.ShapeDtypeStruct((C, 1), jnp.float32),
            jax.ShapeDtypeStruct((C, 2), jnp.float32),
        ],
    )(tf, rows, t_w, t_p, t_aw, t_ab, epw1, epb1, epw2, epb2,
      ohw1, ohb1, ohw2, ohb2)


def kernel(x, edge_index, first_edge, candidate_edges, t,
           gin_W1, gin_b1, gin_W2, gin_b2, gin_eps, bn_gamma, bn_beta,
           t_W, t_P, t_aff_w, t_aff_b,
           ep_W1, ep_b1, ep_W2, ep_b2, oh_W1, oh_b1, oh_W2, oh_b2):
    eidx = jnp.stack([edge_index[0].reshape(NW, NCHUNK, CH),
                      edge_index[1].reshape(NW, NCHUNK, CH)], axis=2)
    eidx = jnp.pad(eidx, ((0, 0), (0, 1), (0, 0), (0, 0)))

    h = x
    for l in range(L):
        agg = _seg_sum(h, eidx)
        eps1 = (1.0 + gin_eps[l]).reshape(1)
        z, st = _mlp(h, agg, gin_W1[l], gin_b1[l].reshape(1, D),
                     gin_W2[l], gin_b2[l].reshape(1, D), eps1)
        h = _bnres(z, h, st, bn_gamma[l].reshape(1, D), bn_beta[l].reshape(1, D))

    pad1 = (jnp.arange(6, dtype=jnp.int32) * 131) % N
    pad2 = (jnp.arange(G - 1032, dtype=jnp.int32) * 131) % N
    idx = jnp.concatenate([
        first_edge.astype(jnp.int32), pad1,
        candidate_edges[:, 0].astype(jnp.int32),
        candidate_edges[:, 1].astype(jnp.int32), pad2,
    ])
    rows = _gather_rows(h, idx)

    tf = jnp.asarray(t, dtype=jnp.float32).reshape(1)
    partner, orient = _score(
        tf, rows,
        t_W.reshape(1, -1), t_P.reshape(1, -1),
        t_aff_w.reshape(1, -1), t_aff_b.reshape(1, -1),
        ep_W1, ep_b1.reshape(1, D), ep_W2.reshape(D, 1),
        ep_b2.reshape(1, 1), oh_W1, oh_b1.reshape(1, D),
        oh_W2, oh_b2.reshape(1, 2))
    return (partner.reshape(C), orient)
